# baseline (speedup 1.0000x reference)
import functools

import jax
import jax.numpy as jnp
from jax import lax
from jax.experimental import pallas as pl
from jax.experimental.pallas import tpu as pltpu
from jax.experimental.pallas import tpu_sc as plsc

K = 16
NPG = 100
NPAD = 112
NGRP = NPAD // 16
BIG = 1 << 30
INV_DEG = 1.0 / 17.0
NWORKERS = 32
FW = 17


def _mm1_kernel(x_ref, w_ref, o_ref):
    ng = o_ref.shape[0]
    xw = jnp.dot(x_ref[...], w_ref[...], preferred_element_type=jnp.float32)
    xw = jnp.concatenate(
        [xw, jnp.zeros((xw.shape[0], FW - xw.shape[1]), jnp.float32)], axis=1)
    o_ref[:, :NPG, :] = xw.reshape(ng, NPG, FW)
    o_ref[:, NPG:, :] = jnp.zeros((ng, NPAD - NPG, FW), jnp.float32)


def _mm2_kernel(g_ref, w_ref, b_ref, o_ref):
    d_hid = w_ref.shape[0]
    o_ref[...] = (jnp.dot(g_ref[..., :d_hid], w_ref[...],
                          preferred_element_type=jnp.float32) * INV_DEG
                  + b_ref[...])


def _sc_body(coords_hbm, xw1_hbm, b1b_hbm, g2_hbm,
             c0, c1, w0, w1, h_v, o0, o1, idxT_v, b1b_v,
             sc0, sc1, sw0, sw1, so0, so1):
    B = coords_hbm.shape[0]
    wid = lax.axis_index("s") * 2 + lax.axis_index("c")

    pltpu.sync_copy(b1b_hbm, b1b_v)
    lane = lax.iota(jnp.int32, 16)

    for gi0 in range(NGRP):
        z16 = jnp.zeros((16,), jnp.float32)
        plsc.store_scatter(o0, [(lane + 16 * gi0) * FW + K], z16)
        plsc.store_scatter(o1, [(lane + 16 * gi0) * FW + K], z16)
    for n in range(K):
        idxT_v[n, pl.ds(NPG - 4, 16)] = jnp.zeros((16,), jnp.int32)

    def _merge(a, b):
        ak, av = a
        bk, bv = b
        bk2 = lax.rev(bk, (0,))
        bv2 = lax.rev(bv, (0,))
        ta = ak <= bk2
        ck = jnp.where(ta, ak, bk2)
        cv = jnp.where(ta, av, bv2)
        return plsc.sort_key_val(ck, cv)

    def _pass1(cs):
        @plsc.parallel_loop(0, NPG, 1, unroll=4)
        def _node(i):
            ii = jnp.full((16,), i, jnp.int32)
            xi = plsc.load_gather(cs, [ii])
            yi = plsc.load_gather(cs, [ii + NPAD])
            groups = []
            for j in range(NGRP):
                xg = cs[pl.ds(16 * j, 16)]
                yg = cs[pl.ds(NPAD + 16 * j, 16)]
                dx = xg - xi
                dy = yg - yi
                d = dx * dx + dy * dy
                jv = lane + (16 * j)
                key = d * 128 + jv
                if 16 * (j + 1) > NPG:
                    key = jnp.where(jv >= NPG, BIG, key)
                key = jnp.where(jv == i, BIG, key)
                groups.append(plsc.sort_key_val(key, jv))
            m01 = _merge(groups[0], groups[1])
            m23 = _merge(groups[2], groups[3])
            m45 = _merge(groups[4], groups[5])
            m0123 = _merge(m01, m23)
            ak, av = _merge(m45, groups[6])
            bk = lax.rev(ak, (0,))
            bv = lax.rev(av, (0,))
            ta = m0123[0] <= bk
            va = jnp.where(ta, m0123[1], bv)
            plsc.store_scatter(idxT_v, [lane, ii], va)

    def _agg(gi, src_v, dst_v, relu_bias):
        base = gi * 16
        nflat = (lane + base) * FW
        accs = [plsc.load_gather(src_v, [nflat + f]) for f in range(K)]
        for n in range(K):
            ib = idxT_v[n, pl.ds(base, 16)] * FW
            for f in range(K):
                accs[f] = accs[f] + plsc.load_gather(src_v, [ib + f])
        for f in range(K):
            v = accs[f]
            if relu_bias:
                v = jnp.maximum(v * INV_DEG + b1b_v[f], 0.0)
            plsc.store_scatter(dst_v, [nflat + f], v)
        return 0

    def _fire_in(g, cs, ws, sc, sw):
        @pl.when(g < B)
        def _():
            pltpu.async_copy(coords_hbm.at[g], cs, sc)
            pltpu.async_copy(xw1_hbm.at[g], ws, sw)

    def _sched_step(g, cs, ws, os, sc, sw, so, cs_n, ws_n, sc_n, sw_n):
        @pl.when(g < B)
        def _():
            pltpu.make_async_copy(coords_hbm.at[g], cs, sc).wait()
            _fire_in(g + NWORKERS, cs_n, ws_n, sc_n, sw_n)
            _pass1(cs)
            pltpu.make_async_copy(xw1_hbm.at[g], ws, sw).wait()

            @pl.when(g >= 2 * NWORKERS)
            def _():
                pltpu.make_async_copy(os, g2_hbm.at[g], so).wait()

            lax.fori_loop(0, NGRP, lambda gi, c: _agg(gi, ws, h_v, True), 0)
            lax.fori_loop(0, NGRP, lambda gi, c: _agg(gi, h_v, os, False), 0)
            pltpu.async_copy(os, g2_hbm.at[g], so)

    _fire_in(wid, c0, w0, sc0, sw0)

    def _pair(t2, _):
        g0 = wid + (2 * NWORKERS) * t2
        _sched_step(g0, c0, w0, o0, sc0, sw0, so0, c1, w1, sc1, sw1)
        _sched_step(g0 + NWORKERS, c1, w1, o1, sc1, sw1, so1, c0, w0, sc0, sw0)
        return 0

    n_steps = (B + NWORKERS - 1) // NWORKERS
    lax.fori_loop(0, (n_steps + 1) // 2, _pair, 0)

    pltpu.make_async_copy(o0, g2_hbm.at[wid], so0).wait()
    pltpu.make_async_copy(o1, g2_hbm.at[wid], so1).wait()


@jax.jit
def kernel(coo, x, W1, b1, W2, b2):
    N = x.shape[0]
    B = N // NPG
    d_in = x.shape[1]
    d_hid = W1.shape[1]
    d_out = W2.shape[1]

    blk = 2000
    gpb = blk // NPG
    xw1r3 = pl.pallas_call(
        _mm1_kernel,
        grid=(N // blk,),
        in_specs=[
            pl.BlockSpec((blk, d_in), lambda i: (i, 0)),
            pl.BlockSpec((d_in, d_hid), lambda i: (0, 0)),
        ],
        out_specs=pl.BlockSpec((gpb, NPAD, FW), lambda i: (i, 0, 0)),
        out_shape=jax.ShapeDtypeStruct((B, NPAD, FW), jnp.float32),
    )(x, W1)

    coords = jnp.zeros((B, 2 * NPAD), jnp.int32)
    coords = coords.at[:, :NPG].set(coo[:, 0].reshape(B, NPG))
    coords = coords.at[:, NPAD:NPAD + NPG].set(coo[:, 1].reshape(B, NPG))
    xw1r = xw1r3.reshape(B, NPAD * FW)
    b1b = jnp.broadcast_to(b1[:, None], (d_hid, 16))

    mesh = plsc.VectorSubcoreMesh(core_axis_name="c", subcore_axis_name="s",
                                  num_cores=2, num_subcores=16)
    g2 = pl.kernel(
        _sc_body,
        out_type=jax.ShapeDtypeStruct((B, NPAD * FW), jnp.float32),
        mesh=mesh,
        compiler_params=pltpu.CompilerParams(needs_layout_passes=False),
        scratch_types=[
            pltpu.VMEM((2 * NPAD,), jnp.int32),
            pltpu.VMEM((2 * NPAD,), jnp.int32),
            pltpu.VMEM((NPAD * FW,), jnp.float32),
            pltpu.VMEM((NPAD * FW,), jnp.float32),
            pltpu.VMEM((NPAD * FW,), jnp.float32),
            pltpu.VMEM((NPAD * FW,), jnp.float32),
            pltpu.VMEM((NPAD * FW,), jnp.float32),
            pltpu.VMEM((K, NPAD + 1), jnp.int32),
            pltpu.VMEM((d_hid, 16), jnp.float32),
            pltpu.SemaphoreType.DMA,
            pltpu.SemaphoreType.DMA,
            pltpu.SemaphoreType.DMA,
            pltpu.SemaphoreType.DMA,
            pltpu.SemaphoreType.DMA,
            pltpu.SemaphoreType.DMA,
        ],
    )(coords, xw1r, b1b)

    out = pl.pallas_call(
        _mm2_kernel,
        grid=(N // blk,),
        in_specs=[
            pl.BlockSpec((blk, FW), lambda i: (i, 0)),
            pl.BlockSpec((d_hid, d_out), lambda i: (0, 0)),
            pl.BlockSpec((1, d_out), lambda i: (0, 0)),
        ],
        out_specs=pl.BlockSpec((blk, d_out), lambda i: (i, 0)),
        out_shape=jax.ShapeDtypeStruct((N, d_out), jnp.float32),
    )(g2.reshape(B, NPAD, FW)[:, :NPG].reshape(N, FW), W2, b2[None])
    return out


# speedup vs baseline: 1.0231x; 1.0231x over previous
import functools

import jax
import jax.numpy as jnp
from jax import lax
from jax.experimental import pallas as pl
from jax.experimental.pallas import tpu as pltpu
from jax.experimental.pallas import tpu_sc as plsc

K = 16
NPG = 100
NPAD = 112
NGRP = NPAD // 16
BIG = 1 << 30
INV_DEG = 1.0 / 17.0
NWORKERS = 32
FW = 17


def _mm1_kernel(x_ref, w_ref, o_ref):
    ng = o_ref.shape[0]
    xw = jnp.dot(x_ref[...], w_ref[...], preferred_element_type=jnp.float32)
    xw = jnp.concatenate(
        [xw, jnp.zeros((xw.shape[0], FW - xw.shape[1]), jnp.float32)], axis=1)
    o_ref[:, :NPG, :] = xw.reshape(ng, NPG, FW)
    o_ref[:, NPG:, :] = jnp.zeros((ng, NPAD - NPG, FW), jnp.float32)


def _mm2_kernel(g_ref, w_ref, b_ref, o_ref):
    d_hid = w_ref.shape[0]
    o_ref[...] = (jnp.dot(g_ref[..., :d_hid], w_ref[...],
                          preferred_element_type=jnp.float32) * INV_DEG
                  + b_ref[...])


def _sc_body(coords_hbm, xw1_hbm, b1b_hbm, g2_hbm,
             c0, c1, w0, w1, h_v, o0, o1, idxT_v, b1b_v,
             sc0, sc1, sw0, sw1, so0, so1):
    B = coords_hbm.shape[0]
    wid = lax.axis_index("s") * 2 + lax.axis_index("c")

    pltpu.sync_copy(b1b_hbm, b1b_v)
    lane = lax.iota(jnp.int32, 16)

    for gi0 in range(NGRP):
        z16 = jnp.zeros((16,), jnp.float32)
        plsc.store_scatter(o0, [(lane + 16 * gi0) * FW + K], z16)
        plsc.store_scatter(o1, [(lane + 16 * gi0) * FW + K], z16)
    for n in range(K):
        idxT_v[n, pl.ds(NPG - 4, 16)] = jnp.zeros((16,), jnp.int32)

    def _merge(a, b):
        ak, av = a
        bk, bv = b
        bk2 = lax.rev(bk, (0,))
        bv2 = lax.rev(bv, (0,))
        ta = ak <= bk2
        ck = jnp.where(ta, ak, bk2)
        cv = jnp.where(ta, av, bv2)
        return plsc.sort_key_val(ck, cv)

    def _pass1(cs):
        @plsc.parallel_loop(0, NPG, 1, unroll=2)
        def _node(i):
            ii = jnp.full((16,), i, jnp.int32)
            xi = plsc.load_gather(cs, [ii])
            yi = plsc.load_gather(cs, [ii + NPAD])
            groups = []
            for j in range(NGRP):
                xg = cs[pl.ds(16 * j, 16)]
                yg = cs[pl.ds(NPAD + 16 * j, 16)]
                dx = xg - xi
                dy = yg - yi
                d = dx * dx + dy * dy
                jv = lane + (16 * j)
                key = d * 128 + jv
                if 16 * (j + 1) > NPG:
                    key = jnp.where(jv >= NPG, BIG, key)
                key = jnp.where(jv == i, BIG, key)
                groups.append(plsc.sort_key_val(key, jv))
            m01 = _merge(groups[0], groups[1])
            m23 = _merge(groups[2], groups[3])
            m45 = _merge(groups[4], groups[5])
            m0123 = _merge(m01, m23)
            ak, av = _merge(m45, groups[6])
            bk = lax.rev(ak, (0,))
            bv = lax.rev(av, (0,))
            ta = m0123[0] <= bk
            va = jnp.where(ta, m0123[1], bv)
            plsc.store_scatter(idxT_v, [lane, ii], va)

    def _agg(gi, src_v, dst_v, relu_bias):
        base = gi * 16
        nflat = (lane + base) * FW
        accs = [plsc.load_gather(src_v, [nflat + f]) for f in range(K)]
        for n in range(K):
            ib = idxT_v[n, pl.ds(base, 16)] * FW
            for f in range(K):
                accs[f] = accs[f] + plsc.load_gather(src_v, [ib + f])
        for f in range(K):
            v = accs[f]
            if relu_bias:
                v = jnp.maximum(v * INV_DEG + b1b_v[f], 0.0)
            plsc.store_scatter(dst_v, [nflat + f], v)
        return 0

    def _fire_in(g, cs, ws, sc, sw):
        @pl.when(g < B)
        def _():
            pltpu.async_copy(coords_hbm.at[g], cs, sc)
            pltpu.async_copy(xw1_hbm.at[g], ws, sw)

    def _sched_step(g, cs, ws, os, sc, sw, so, cs_n, ws_n, sc_n, sw_n):
        @pl.when(g < B)
        def _():
            pltpu.make_async_copy(coords_hbm.at[g], cs, sc).wait()
            _fire_in(g + NWORKERS, cs_n, ws_n, sc_n, sw_n)
            _pass1(cs)
            pltpu.make_async_copy(xw1_hbm.at[g], ws, sw).wait()

            @pl.when(g >= 2 * NWORKERS)
            def _():
                pltpu.make_async_copy(os, g2_hbm.at[g], so).wait()

            lax.fori_loop(0, NGRP, lambda gi, c: _agg(gi, ws, h_v, True), 0)
            lax.fori_loop(0, NGRP, lambda gi, c: _agg(gi, h_v, os, False), 0)
            pltpu.async_copy(os, g2_hbm.at[g], so)

    _fire_in(wid, c0, w0, sc0, sw0)

    def _pair(t2, _):
        g0 = wid + (2 * NWORKERS) * t2
        _sched_step(g0, c0, w0, o0, sc0, sw0, so0, c1, w1, sc1, sw1)
        _sched_step(g0 + NWORKERS, c1, w1, o1, sc1, sw1, so1, c0, w0, sc0, sw0)
        return 0

    n_steps = (B + NWORKERS - 1) // NWORKERS
    lax.fori_loop(0, (n_steps + 1) // 2, _pair, 0)

    pltpu.make_async_copy(o0, g2_hbm.at[wid], so0).wait()
    pltpu.make_async_copy(o1, g2_hbm.at[wid], so1).wait()


@jax.jit
def kernel(coo, x, W1, b1, W2, b2):
    N = x.shape[0]
    B = N // NPG
    d_in = x.shape[1]
    d_hid = W1.shape[1]
    d_out = W2.shape[1]

    blk = 2000
    gpb = blk // NPG
    xw1r3 = pl.pallas_call(
        _mm1_kernel,
        grid=(N // blk,),
        in_specs=[
            pl.BlockSpec((blk, d_in), lambda i: (i, 0)),
            pl.BlockSpec((d_in, d_hid), lambda i: (0, 0)),
        ],
        out_specs=pl.BlockSpec((gpb, NPAD, FW), lambda i: (i, 0, 0)),
        out_shape=jax.ShapeDtypeStruct((B, NPAD, FW), jnp.float32),
    )(x, W1)

    coords = jnp.zeros((B, 2 * NPAD), jnp.int32)
    coords = coords.at[:, :NPG].set(coo[:, 0].reshape(B, NPG))
    coords = coords.at[:, NPAD:NPAD + NPG].set(coo[:, 1].reshape(B, NPG))
    xw1r = xw1r3.reshape(B, NPAD * FW)
    b1b = jnp.broadcast_to(b1[:, None], (d_hid, 16))

    mesh = plsc.VectorSubcoreMesh(core_axis_name="c", subcore_axis_name="s",
                                  num_cores=2, num_subcores=16)
    g2 = pl.kernel(
        _sc_body,
        out_type=jax.ShapeDtypeStruct((B, NPAD * FW), jnp.float32),
        mesh=mesh,
        compiler_params=pltpu.CompilerParams(needs_layout_passes=False),
        scratch_types=[
            pltpu.VMEM((2 * NPAD,), jnp.int32),
            pltpu.VMEM((2 * NPAD,), jnp.int32),
            pltpu.VMEM((NPAD * FW,), jnp.float32),
            pltpu.VMEM((NPAD * FW,), jnp.float32),
            pltpu.VMEM((NPAD * FW,), jnp.float32),
            pltpu.VMEM((NPAD * FW,), jnp.float32),
            pltpu.VMEM((NPAD * FW,), jnp.float32),
            pltpu.VMEM((K, NPAD + 1), jnp.int32),
            pltpu.VMEM((d_hid, 16), jnp.float32),
            pltpu.SemaphoreType.DMA,
            pltpu.SemaphoreType.DMA,
            pltpu.SemaphoreType.DMA,
            pltpu.SemaphoreType.DMA,
            pltpu.SemaphoreType.DMA,
            pltpu.SemaphoreType.DMA,
        ],
    )(coords, xw1r, b1b)

    out = pl.pallas_call(
        _mm2_kernel,
        grid=(N // blk,),
        in_specs=[
            pl.BlockSpec((blk, FW), lambda i: (i, 0)),
            pl.BlockSpec((d_hid, d_out), lambda i: (0, 0)),
            pl.BlockSpec((1, d_out), lambda i: (0, 0)),
        ],
        out_specs=pl.BlockSpec((blk, d_out), lambda i: (i, 0)),
        out_shape=jax.ShapeDtypeStruct((N, d_out), jnp.float32),
    )(g2.reshape(B, NPAD, FW)[:, :NPG].reshape(N, FW), W2, b2[None])
    return out
